# baseline (speedup 1.0000x reference)
import numpy as np
import jax
import jax.numpy as jnp
from jax import lax
from jax.experimental import pallas as pl
from jax.experimental.pallas import tpu as pltpu

_B, _T, _N, _F = 2, 8, 10000, 16
_H = 128
_G = 100
_BN = _B * _N
_R = 5
_NB = _G // _R
_F32 = jnp.float32
_OFFS = [(-1, -1), (-1, 0), (-1, 1), (0, -1), (0, 1), (1, -1), (1, 0), (1, 1)]


def _col_roll(a, dc):
    if dc == 0:
        return a
    s = (-dc) % _G
    return jnp.concatenate([a[:, :, _G - s:], a[:, :, :_G - s]], axis=2)


def _shift3(a, dr, dc):
    sr = (-dr) % _G
    if sr:
        a = jnp.concatenate([a[:, _G - sr:], a[:, :_G - sr]], axis=1)
    sc = (-dc) % _G
    if sc:
        a = jnp.concatenate([a[:, :, _G - sc:], a[:, :, :_G - sc]], axis=2)
    return a


def _dot(lhs, rhs, cdim):
    return lax.dot_general(lhs, rhs, (((cdim,), (0,)), ((), ())),
                           preferred_element_type=_F32)


def _stgnn(x5_ref,
           ws0, wd0, we0, wr0, at0, bs0,
           ws1, wd1, we1, wr1, at1, bs1,
           wih, whh, bih, bhh, wo, bo,
           y_ref, Ag, Hc, Hg, Ep):
    t = pl.program_id(0)
    u = x5_ref[0, :, :, 1]
    v = x5_ref[0, :, :, 2]

    for k, (dr, dc) in enumerate(_OFFS):
        nrm = float(np.sqrt(dr * dr + dc * dc))
        d0, d1 = -dc / nrm, -dr / nrm
        us = _shift3(u, dr, dc)
        vs = _shift3(v, dr, dc)
        speed = jnp.sqrt(us * us + vs * vs)
        proj = us * d0 + vs * d1
        eat = (proj / (speed + 1e-6), speed * 0.1, proj * 0.1)
        for j in range(3):
            ch = 3 * k + j
            for i in range(_NB):
                Ep[:, i, :, :, ch:ch + 1] = \
                    eat[j][:, i * _R:(i + 1) * _R][..., None]

    for l, (ws, wd, we, wr, at, bs) in enumerate(
            ((ws0, wd0, we0, wr0, at0, bs0),
             (ws1, wd1, we1, wr1, at1, bs1))):
        if l == 0:
            def pa_body(i, c):
                a = _dot(x5_ref[0, :, pl.ds(i * _R, _R)], ws[...], 2)
                Ag[:, pl.ds(1 + i * _R, _R)] = a
                return c
        else:
            def pa_body(i, c):
                a = _dot(Hc[:, pl.ds(i * _R, _R)], ws[...], 3)
                Ag[:, pl.ds(1 + i * _R, _R)] = a
                return c
        lax.fori_loop(0, _NB, pa_body, 0)
        Ag[:, 0:1] = Ag[:, _G:_G + 1]
        Ag[:, _G + 1:_G + 2] = Ag[:, 1:2]

        wem = we[...]
        atc = at[...]
        bs4 = bs[...].reshape(1, 1, 1, _H)

        def bc_body(i, c, _l=l, _wd=wd, _wr=wr, _wem=wem, _atc=atc,
                    _bs4=bs4):
            if _l == 0:
                hb = x5_ref[0, :, pl.ds(i * _R, _R)]
                dg = _dot(hb, _wd[...], 2)
                rb = _dot(hb, _wr[...], 2) + _bs4
            else:
                hb = Hc[:, pl.ds(i * _R, _R)]
                dg = _dot(hb, _wd[...], 3)
                rb = _dot(hb, _wr[...], 3) + _bs4
            wemv = _wem[...]
            atcv = _atc[...]

            svals = []
            abs_ = []
            for k, (dr, dc) in enumerate(_OFFS):
                ab = _col_roll(Ag[:, pl.ds(i * _R + dr + 1, _R)], dc)
                abs_.append(ab)
                e3 = Ep[:, i, :, :, 3 * k:3 * k + 3]
                he = _dot(e3, wemv, 3)
                z = ab + dg + he
                z = jnp.where(z >= 0, z, 0.2 * z)
                svals.append(lax.dot_general(
                    atcv, z, (((0,), (3,)), ((), ())),
                    preferred_element_type=_F32))

            m = svals[0]
            for s_ in svals[1:]:
                m = jnp.maximum(m, s_)
            exs = [jnp.exp(s_ - m) for s_ in svals]
            den = exs[0]
            for e_ in exs[1:]:
                den = den + e_
            inv = 1.0 / (den + 1e-9)

            accA = None
            ce = None
            for k, (dr, dc) in enumerate(_OFFS):
                ab = abs_[k]
                e3 = Ep[:, i, :, :, 3 * k:3 * k + 3]
                exb = exs[k][0][..., None]
                if k == 0:
                    accA = exb * ab
                    ce = exb * e3
                else:
                    accA = accA + exb * ab
                    ce = ce + exb * e3
            acc = rb + inv[0][..., None] * (accA + _dot(ce, wemv, 3))
            acc = jnp.where(acc > 0, acc,
                            jnp.exp(jnp.minimum(acc, 0.0)) - 1.0)
            Hc[:, pl.ds(i * _R, _R)] = acc
            return c

        lax.fori_loop(0, _NB, bc_body, 0)

    @pl.when(t == 0)
    def _():
        Hg[...] = jnp.zeros((_B, _G, _G, _H), _F32)

    bih4 = bih[...].reshape(1, 1, 1, 3 * _H)
    bhh4 = bhh[...].reshape(1, 1, 1, 3 * _H)

    def gru_body(i, c):
        hsp = Hc[:, pl.ds(i * _R, _R)]
        hprev = Hg[:, pl.ds(i * _R, _R)]
        gi = _dot(hsp, wih[...], 3) + bih4
        gh = _dot(hprev, whh[...], 3) + bhh4
        r = jax.nn.sigmoid(gi[..., :_H] + gh[..., :_H])
        zz = jax.nn.sigmoid(gi[..., _H:2 * _H] + gh[..., _H:2 * _H])
        nn_ = jnp.tanh(gi[..., 2 * _H:] + r * gh[..., 2 * _H:])
        Hg[:, pl.ds(i * _R, _R)] = (1.0 - zz) * nn_ + zz * hprev
        return c

    lax.fori_loop(0, _NB, gru_body, 0)

    @pl.when(t == _T - 1)
    def _():
        wo4 = wo[...].reshape(1, 1, 1, _H)
        bo3 = bo[...].reshape(1, 1, 1)
        for i in range(_NB):
            hnew = Hg[:, i * _R:(i + 1) * _R]
            y_ref[:, i * _R:(i + 1) * _R] = \
                jnp.sum(hnew * wo4, axis=-1) + bo3


def kernel(x, params):
    x5 = jnp.transpose(x, (1, 0, 2, 3)).reshape(_T, _B, _G, _G, _F)
    x5 = jnp.swapaxes(x5, 3, 4)
    c0 = params["conv0"]
    c1 = params["conv1"]
    g = params["gru"]
    ops = [x5,
           c0["W_src"], c0["W_dst"], c0["W_edge"], c0["W_root"],
           c0["att"].reshape(_H, 1), c0["bias"].reshape(1, _H),
           c1["W_src"], c1["W_dst"], c1["W_edge"], c1["W_root"],
           c1["att"].reshape(_H, 1), c1["bias"].reshape(1, _H),
           g["W_ih"].T, g["W_hh"].T,
           g["b_ih"].reshape(1, 3 * _H), g["b_hh"].reshape(1, 3 * _H),
           params["W_out"].reshape(1, _H), params["b_out"].reshape(1, 1)]
    in_specs = [pl.BlockSpec((1, _B, _G, _F, _G),
                             lambda t: (t, 0, 0, 0, 0))]
    for o in ops[1:]:
        in_specs.append(pl.BlockSpec(o.shape, lambda t: (0, 0)))
    y = pl.pallas_call(
        _stgnn,
        grid=(_T,),
        in_specs=in_specs,
        out_specs=pl.BlockSpec((_B, _G, _G), lambda t: (0, 0, 0)),
        out_shape=jax.ShapeDtypeStruct((_B, _G, _G), _F32),
        scratch_shapes=[pltpu.VMEM((_B, _G + 2, _G, _H), _F32),
                        pltpu.VMEM((_B, _G, _G, _H), _F32),
                        pltpu.VMEM((_B, _G, _G, _H), _F32),
                        pltpu.VMEM((_B, _NB, _R, _G, 27), _F32)],
    )(*ops)
    return y.reshape(_B, _N)


# speedup vs baseline: 1.0592x; 1.0592x over previous
import numpy as np
import jax
import jax.numpy as jnp
from jax import lax
from jax.experimental import pallas as pl
from jax.experimental.pallas import tpu as pltpu

_B, _T, _N, _F = 2, 8, 10000, 16
_H = 128
_G = 100
_BN = _B * _N
_R = 4
_NB = _G // _R
_F32 = jnp.float32
_OFFS = [(-1, -1), (-1, 0), (-1, 1), (0, -1), (0, 1), (1, -1), (1, 0), (1, 1)]


def _col_roll(a, dc):
    if dc == 0:
        return a
    s = (-dc) % _G
    return jnp.concatenate([a[:, :, _G - s:], a[:, :, :_G - s]], axis=2)


def _shift3(a, dr, dc):
    sr = (-dr) % _G
    if sr:
        a = jnp.concatenate([a[:, _G - sr:], a[:, :_G - sr]], axis=1)
    sc = (-dc) % _G
    if sc:
        a = jnp.concatenate([a[:, :, _G - sc:], a[:, :, :_G - sc]], axis=2)
    return a


def _dot(lhs, rhs, cdim):
    return lax.dot_general(lhs, rhs, (((cdim,), (0,)), ((), ())),
                           preferred_element_type=_F32)


def _stgnn(x5_ref,
           ws0, wd0, we0, wr0, at0, bs0,
           ws1, wd1, we1, wr1, at1, bs1,
           wih, whh, bih, bhh, wo, bo,
           y_ref, Ag, Hc, Hg, Ep):
    t = pl.program_id(0)
    u = x5_ref[0, :, :, 1]
    v = x5_ref[0, :, :, 2]

    for k, (dr, dc) in enumerate(_OFFS):
        nrm = float(np.sqrt(dr * dr + dc * dc))
        d0, d1 = -dc / nrm, -dr / nrm
        us = _shift3(u, dr, dc)
        vs = _shift3(v, dr, dc)
        speed = jnp.sqrt(us * us + vs * vs)
        proj = us * d0 + vs * d1
        eat = (proj / (speed + 1e-6), speed * 0.1, proj * 0.1)
        for j in range(3):
            ch = 3 * k + j
            for i in range(_NB):
                Ep[:, i, :, :, ch:ch + 1] = \
                    eat[j][:, i * _R:(i + 1) * _R][..., None]

    @pl.when(t == 0)
    def _():
        Hg[...] = jnp.zeros((_B, _G, _G, _H), _F32)

    bih4 = bih[...].reshape(1, 1, 1, 3 * _H)
    bhh4 = bhh[...].reshape(1, 1, 1, 3 * _H)

    for l, (ws, wd, we, wr, at, bs) in enumerate(
            ((ws0, wd0, we0, wr0, at0, bs0),
             (ws1, wd1, we1, wr1, at1, bs1))):
        if l == 0:
            def pa_body(i, c):
                a = _dot(x5_ref[0, :, pl.ds(i * _R, _R)], ws[...], 2)
                Ag[:, pl.ds(1 + i * _R, _R)] = a
                return c
        else:
            def pa_body(i, c):
                a = _dot(Hc[:, pl.ds(i * _R, _R)], ws[...], 3)
                Ag[:, pl.ds(1 + i * _R, _R)] = a
                return c
        lax.fori_loop(0, _NB, pa_body, 0)
        Ag[:, 0:1] = Ag[:, _G:_G + 1]
        Ag[:, _G + 1:_G + 2] = Ag[:, 1:2]

        wem = we[...]
        atc = at[...]
        bs4 = bs[...].reshape(1, 1, 1, _H)

        def bc_body(i, c, _l=l, _wd=wd, _wr=wr, _wem=wem, _atc=atc,
                    _bs4=bs4):
            if _l == 0:
                hb = x5_ref[0, :, pl.ds(i * _R, _R)]
                dg = _dot(hb, _wd[...], 2)
                rb = _dot(hb, _wr[...], 2) + _bs4
            else:
                hb = Hc[:, pl.ds(i * _R, _R)]
                dg = _dot(hb, _wd[...], 3)
                rb = _dot(hb, _wr[...], 3) + _bs4
            wemv = _wem[...]
            atcv = _atc[...]

            svals = []
            abs_ = []
            for k, (dr, dc) in enumerate(_OFFS):
                ab = _col_roll(Ag[:, pl.ds(i * _R + dr + 1, _R)], dc)
                abs_.append(ab)
                e3 = Ep[:, i, :, :, 3 * k:3 * k + 3]
                he = _dot(e3, wemv, 3)
                z = ab + dg + he
                z = jnp.where(z >= 0, z, 0.2 * z)
                svals.append(lax.dot_general(
                    atcv, z, (((0,), (3,)), ((), ())),
                    preferred_element_type=_F32))

            m = svals[0]
            for s_ in svals[1:]:
                m = jnp.maximum(m, s_)
            exs = [jnp.exp(s_ - m) for s_ in svals]
            den = exs[0]
            for e_ in exs[1:]:
                den = den + e_
            inv = 1.0 / (den + 1e-9)

            accA = None
            ce = None
            for k, (dr, dc) in enumerate(_OFFS):
                ab = abs_[k]
                e3 = Ep[:, i, :, :, 3 * k:3 * k + 3]
                exb = exs[k][0][..., None]
                if k == 0:
                    accA = exb * ab
                    ce = exb * e3
                else:
                    accA = accA + exb * ab
                    ce = ce + exb * e3
            acc = rb + inv[0][..., None] * (accA + _dot(ce, wemv, 3))
            acc = jnp.where(acc > 0, acc,
                            jnp.exp(jnp.minimum(acc, 0.0)) - 1.0)
            if _l == 0:
                Hc[:, pl.ds(i * _R, _R)] = acc
            else:
                hprev = Hg[:, pl.ds(i * _R, _R)]
                gi = _dot(acc, wih[...], 3) + bih4
                gh = _dot(hprev, whh[...], 3) + bhh4
                r = jax.nn.sigmoid(gi[..., :_H] + gh[..., :_H])
                zz = jax.nn.sigmoid(gi[..., _H:2 * _H] + gh[..., _H:2 * _H])
                nn_ = jnp.tanh(gi[..., 2 * _H:] + r * gh[..., 2 * _H:])
                Hg[:, pl.ds(i * _R, _R)] = (1.0 - zz) * nn_ + zz * hprev
            return c

        lax.fori_loop(0, _NB, bc_body, 0)

    @pl.when(t == _T - 1)
    def _():
        wo4 = wo[...].reshape(1, 1, 1, _H)
        bo3 = bo[...].reshape(1, 1, 1)
        for i in range(_NB):
            hnew = Hg[:, i * _R:(i + 1) * _R]
            y_ref[:, i * _R:(i + 1) * _R] = \
                jnp.sum(hnew * wo4, axis=-1) + bo3


def kernel(x, params):
    x5 = jnp.transpose(x, (1, 0, 2, 3)).reshape(_T, _B, _G, _G, _F)
    x5 = jnp.swapaxes(x5, 3, 4)
    c0 = params["conv0"]
    c1 = params["conv1"]
    g = params["gru"]
    ops = [x5,
           c0["W_src"], c0["W_dst"], c0["W_edge"], c0["W_root"],
           c0["att"].reshape(_H, 1), c0["bias"].reshape(1, _H),
           c1["W_src"], c1["W_dst"], c1["W_edge"], c1["W_root"],
           c1["att"].reshape(_H, 1), c1["bias"].reshape(1, _H),
           g["W_ih"].T, g["W_hh"].T,
           g["b_ih"].reshape(1, 3 * _H), g["b_hh"].reshape(1, 3 * _H),
           params["W_out"].reshape(1, _H), params["b_out"].reshape(1, 1)]
    in_specs = [pl.BlockSpec((1, _B, _G, _F, _G),
                             lambda t: (t, 0, 0, 0, 0))]
    for o in ops[1:]:
        in_specs.append(pl.BlockSpec(o.shape, lambda t: (0, 0)))
    y = pl.pallas_call(
        _stgnn,
        grid=(_T,),
        in_specs=in_specs,
        out_specs=pl.BlockSpec((_B, _G, _G), lambda t: (0, 0, 0)),
        out_shape=jax.ShapeDtypeStruct((_B, _G, _G), _F32),
        scratch_shapes=[pltpu.VMEM((_B, _G + 2, _G, _H), _F32),
                        pltpu.VMEM((_B, _G, _G, _H), _F32),
                        pltpu.VMEM((_B, _G, _G, _H), _F32),
                        pltpu.VMEM((_B, _NB, _R, _G, 27), _F32)],
    )(*ops)
    return y.reshape(_B, _N)
